# baseline (speedup 1.0000x reference)
import functools

import jax
import jax.numpy as jnp
from jax import lax
from jax.experimental import pallas as pl
from jax.experimental.pallas import tpu as pltpu
from jax.experimental.pallas import tpu_sc as plsc

_info = plsc.get_sparse_core_info()
_NC, _NS, _L = _info.num_cores, _info.num_subcores, _info.num_lanes


def _make_sc_factors(B, C):
    per_sub = B // _NS
    per_w = B // (_NC * _NS)
    n_cnt_chunks = per_sub // _L
    n_out_chunks = per_w // _L

    mesh = plsc.VectorSubcoreMesh(core_axis_name="c", subcore_axis_name="s")

    @functools.partial(
        pl.kernel,
        mesh=mesh,
        out_type=jax.ShapeDtypeStruct((B,), jnp.float32),
        scratch_types=[
            pltpu.VMEM((per_sub,), jnp.int32),
            pltpu.VMEM((per_w,), jnp.float32),
            pltpu.VMEM((_L,), jnp.float32),
            pltpu.VMEM((_L,), jnp.float32),
            pltpu.VMEM((_NS * _L,), jnp.float32),
            pltpu.VMEM_SHARED((_NS * _L,), jnp.float32),
        ],
    )
    def sc_factors(ids_hbm, tr_hbm, s_hbm, ids_v, s_v, tr_v, cnt_v, all_v, shared):
        cid = lax.axis_index("c")
        sid = lax.axis_index("s")
        pltpu.sync_copy(ids_hbm.at[pl.ds(sid * per_sub, per_sub)], ids_v)
        pltpu.sync_copy(tr_hbm, tr_v)

        def cbody(j, accs):
            v = ids_v[pl.ds(j * _L, _L)]
            one = jnp.ones((_L,), jnp.int32)
            zro = jnp.zeros((_L,), jnp.int32)
            return tuple(a + jnp.where(v == c, one, zro) for c, a in enumerate(accs))

        zero = jnp.zeros((_L,), jnp.int32)
        accs = lax.fori_loop(0, n_cnt_chunks, cbody, (zero,) * C)
        lane = lax.iota(jnp.int32, _L)

        def lane_sum(x):
            for k in (1, 2, 4, 8):
                x = x + x.at[lane ^ k].get(mode="promise_in_bounds")
            return x

        cntvec = jnp.zeros((_L,), jnp.float32)
        for c in range(C):
            cnt_c = lane_sum(accs[c]).astype(jnp.float32)
            cntvec = jnp.where(lane == c, cnt_c, cntvec)

        cnt_v[...] = cntvec
        pltpu.sync_copy(cnt_v, shared.at[pl.ds(sid * _L, _L)])
        plsc.subcore_barrier()
        pltpu.sync_copy(shared, all_v)

        def tbody(j, tot):
            return tot + all_v[pl.ds(j * _L, _L)]

        totals = lax.fori_loop(0, _NS, tbody, jnp.zeros((_L,), jnp.float32))

        cur = totals / float(B)
        sfvec = tr_v[...] / cur

        base = cid * per_w

        def gbody(j, _):
            v = ids_v[pl.ds(base + j * _L, _L)]
            s_v[pl.ds(j * _L, _L)] = sfvec.at[v].get(
                mode="promise_in_bounds"
            )
            return 0

        lax.fori_loop(0, n_out_chunks, gbody, 0)
        wid = sid * _NC + cid
        pltpu.sync_copy(s_v, s_hbm.at[pl.ds(wid * per_w, per_w)])

    return sc_factors


def _tc_apply(s_ref, x_ref, out_ref):
    x = x_ref[...]
    s = s_ref[...]
    xs = x * s
    out_ref[...] = xs + (x - xs)


def kernel(input, target_ratios, class_ids):
    B, D = input.shape
    C = target_ratios.shape[0]
    ids = class_ids.astype(jnp.int32)
    tr_pad = jnp.zeros((_L,), jnp.float32).at[:C].set(target_ratios)

    s = _make_sc_factors(B, C)(ids, tr_pad)
    s_col = s.reshape(B, 1)

    R = 8192
    grid = (B // R,)
    return pl.pallas_call(
        _tc_apply,
        grid=grid,
        in_specs=[
            pl.BlockSpec((R, 1), lambda i: (i, 0)),
            pl.BlockSpec((R, D), lambda i: (i, 0)),
        ],
        out_specs=pl.BlockSpec((R, D), lambda i: (i, 0)),
        out_shape=jax.ShapeDtypeStruct((B, D), jnp.float32),
    )(s_col, input)


# speedup vs baseline: 1.2032x; 1.2032x over previous
import functools

import jax
import jax.numpy as jnp
from jax import lax
from jax.experimental import pallas as pl
from jax.experimental.pallas import tpu as pltpu
from jax.experimental.pallas import tpu_sc as plsc

_info = plsc.get_sparse_core_info()
_NC, _NS, _L = _info.num_cores, _info.num_subcores, _info.num_lanes


def _make_sc_counts(B, C):
    per_sub = B // _NS
    n_cnt_chunks = per_sub // _L

    mesh = plsc.VectorSubcoreMesh(core_axis_name="c", subcore_axis_name="s")

    @functools.partial(
        pl.kernel,
        mesh=mesh,
        out_type=jax.ShapeDtypeStruct((_L,), jnp.float32),
        scratch_types=[
            pltpu.VMEM((per_sub,), jnp.int32),
            pltpu.VMEM((_L,), jnp.float32),
            pltpu.VMEM((_NS * _L,), jnp.float32),
            pltpu.VMEM_SHARED((_NS * _L,), jnp.float32),
        ],
    )
    def sc_counts(ids_hbm, cnt_hbm, ids_v, cnt_v, all_v, shared):
        cid = lax.axis_index("c")
        sid = lax.axis_index("s")
        pltpu.sync_copy(ids_hbm.at[pl.ds(sid * per_sub, per_sub)], ids_v)

        def cbody(j, accs):
            v = ids_v[pl.ds(j * _L, _L)]
            one = jnp.ones((_L,), jnp.int32)
            zro = jnp.zeros((_L,), jnp.int32)
            return tuple(a + jnp.where(v == c, one, zro) for c, a in enumerate(accs))

        zero = jnp.zeros((_L,), jnp.int32)
        accs = lax.fori_loop(0, n_cnt_chunks, cbody, (zero,) * C)
        lane = lax.iota(jnp.int32, _L)

        def lane_sum(x):
            for k in (1, 2, 4, 8):
                x = x + x.at[lane ^ k].get(mode="promise_in_bounds")
            return x

        cntvec = jnp.zeros((_L,), jnp.float32)
        for c in range(C):
            cnt_c = lane_sum(accs[c]).astype(jnp.float32)
            cntvec = jnp.where(lane == c, cnt_c, cntvec)

        cnt_v[...] = cntvec
        pltpu.sync_copy(cnt_v, shared.at[pl.ds(sid * _L, _L)])
        plsc.subcore_barrier()
        pltpu.sync_copy(shared, all_v)

        def tbody(j, tot):
            return tot + all_v[pl.ds(j * _L, _L)]

        totals = lax.fori_loop(0, _NS, tbody, jnp.zeros((_L,), jnp.float32))

        @pl.when(jnp.logical_and(cid == 0, sid == 0))
        def _write():
            cnt_v[...] = totals
            pltpu.sync_copy(cnt_v, cnt_hbm)

    return sc_counts


def _tc_apply(B, counts_ref, tr_ref, ids_col_ref, x_ref, out_ref, sf_ref):
    i = pl.program_id(0)
    C = tr_ref.shape[0]

    @pl.when(i == 0)
    def _factors():
        for c in range(C):
            cur_ratio = counts_ref[c] / float(B)
            sf_ref[c] = tr_ref[c] / cur_ratio

    ids_col = ids_col_ref[...]
    s = jnp.full(ids_col.shape, sf_ref[C - 1], dtype=jnp.float32)
    for c in range(C - 2, -1, -1):
        s = jnp.where(ids_col == c, sf_ref[c], s)
    x = x_ref[...]
    xs = x * s
    out_ref[...] = xs + (x - xs)


def kernel(input, target_ratios, class_ids):
    B, D = input.shape
    C = target_ratios.shape[0]
    ids = class_ids.astype(jnp.int32)

    counts = _make_sc_counts(B, C)(ids)
    ids_col = ids.reshape(B, 1)

    R = 8192
    grid = (B // R,)
    return pl.pallas_call(
        functools.partial(_tc_apply, B),
        grid=grid,
        in_specs=[
            pl.BlockSpec(memory_space=pltpu.SMEM),
            pl.BlockSpec(memory_space=pltpu.SMEM),
            pl.BlockSpec((R, 1), lambda i: (i, 0)),
            pl.BlockSpec((R, D), lambda i: (i, 0)),
        ],
        out_specs=pl.BlockSpec((R, D), lambda i: (i, 0)),
        out_shape=jax.ShapeDtypeStruct((B, D), jnp.float32),
        scratch_shapes=[pltpu.SMEM((C,), jnp.float32)],
    )(counts, target_ratios, ids_col, input)
